# baseline (speedup 1.0000x reference)
import functools
import math

import jax
import jax.numpy as jnp
from jax import lax
from jax.experimental import pallas as pl
from jax.experimental.pallas import tpu as pltpu
from jax.experimental.pallas import tpu_sc as plsc

B, N, K, D = 4, 2048, 2048, 4
BW = 0.1
HALF_INV_BW2 = 0.5 / (BW * BW)
INV_BW2 = 1.0 / (BW * BW)
LOGCONST = D * (0.5 * math.log(2.0 * math.pi) + math.log(BW))

NC, NS, L = 2, 16, 16
NW = NC * NS
CPB = NW // B

N_SC = 1024
N_TC = N - N_SC
TN = 512

NPW = N_SC // CPB
NV = NPW // L
BLK = 4

_SQRT2 = math.sqrt(2.0)
_LN2 = math.log(2.0)


def _vlog(a):
    bits = lax.bitcast_convert_type(a, jnp.int32)
    e = lax.shift_right_logical(bits, 23) - 127
    mant = lax.bitcast_convert_type(
        lax.bitwise_or(lax.bitwise_and(bits, 0x007FFFFF), 0x3F800000),
        jnp.float32,
    )
    big = mant > _SQRT2
    mant = jnp.where(big, mant * 0.5, mant)
    e = jnp.where(big, e + 1, e)
    r = (mant - 1.0) / (mant + 1.0)
    r2 = r * r
    poly = 2.0 * r * (1.0 + r2 * (1.0 / 3.0 + r2 * (1.0 / 5.0 + r2 * (1.0 / 7.0 + r2 * (1.0 / 9.0)))))
    return e.astype(jnp.float32) * _LN2 + poly


def _sc_body(st_ref, loct_ref, w_ref, out_ref, locv, wv, qv, sv, outv):
    wid = lax.axis_index("s") * NC + lax.axis_index("c")
    b = wid // CPB

    pltpu.sync_copy(loct_ref.at[b], locv)
    pltpu.sync_copy(w_ref.at[b], wv)
    pltpu.sync_copy(st_ref.at[wid], sv)

    def qbody(j, _):
        sl = pl.ds(j * L, L)
        tds = [locv[pl.ds(d * K + j * L, L)] for d in range(D)]
        acc = tds[0] * tds[0]
        for d in range(1, D):
            acc = acc + tds[d] * tds[d]
        qv[sl] = wv[sl] - HALF_INV_BW2 * acc - LOGCONST
        for d in range(D):
            locv[pl.ds(d * K + j * L, L)] = INV_BW2 * tds[d]
        return 0

    lax.fori_loop(0, K // L, qbody, 0, unroll=2)

    for blk in range(NV // BLK):
        s_regs = []
        p_regs = []
        for v in range(BLK):
            off = (blk * BLK + v) * L
            sd = [sv[pl.ds(d * NPW + off, L)] for d in range(D)]
            s_regs.append(sd)
            nrm = sd[0] * sd[0]
            for d in range(1, D):
                nrm = nrm + sd[d] * sd[d]
            p_regs.append(-HALF_INV_BW2 * nrm)

        neg = jnp.full((L,), -1e30, dtype=jnp.float32)
        zero = jnp.zeros((L,), dtype=jnp.float32)
        init = tuple([neg] * BLK + [zero] * BLK)

        GRP = 4

        def kbody(j, carry):
            ms = list(carry[:BLK])
            accs = list(carry[BLK:])
            lvecs = [locv[pl.ds(d * K + j * L, L)] for d in range(D)]
            qvec = qv[pl.ds(j * L, L)]
            for g in range(L // GRP):
                xs = [[None] * GRP for _ in range(BLK)]
                for ii in range(GRP):
                    i = g * GRP + ii
                    l = [lvecs[d][i] for d in range(D)]
                    qk = qvec[i]
                    for v in range(BLK):
                        sd = s_regs[v]
                        dot = sd[0] * l[0]
                        for d in range(1, D):
                            dot = dot + sd[d] * l[d]
                        xs[v][ii] = (p_regs[v] + qk) + dot
                for v in range(BLK):
                    mn = ms[v]
                    for ii in range(GRP):
                        mn = jnp.maximum(mn, xs[v][ii])
                    e01 = jnp.exp(xs[v][0] - mn) + jnp.exp(xs[v][1] - mn)
                    e23 = jnp.exp(xs[v][2] - mn) + jnp.exp(xs[v][3] - mn)
                    accs[v] = accs[v] * jnp.exp(ms[v] - mn) + (e01 + e23)
                    ms[v] = mn
            return tuple(ms + accs)

        res = lax.fori_loop(0, K // L, kbody, init, unroll=1)
        for v in range(BLK):
            off = (blk * BLK + v) * L
            outv[pl.ds(off, L)] = res[v] + _vlog(res[BLK + v])

    pltpu.sync_copy(outv, out_ref.at[wid])


DP = 8


def _tc_body(s_ref, loct_ref, w_ref, out_ref):
    s = s_ref[0]
    lt = loct_ref[0]
    q = w_ref[0, 0] - HALF_INV_BW2 * jnp.sum(lt * lt, axis=0) - LOGCONST
    p = -HALF_INV_BW2 * jnp.sum(s * s, axis=1)
    dot = lax.dot_general(
        s,
        lt,
        (((1,), (0,)), ((), ())),
        preferred_element_type=jnp.float32,
        precision=lax.Precision.HIGHEST,
    )
    x = q[None, :] + p[:, None] + INV_BW2 * dot
    m = jnp.max(x, axis=1)
    acc = jnp.sum(jnp.exp(x - m[:, None]), axis=1)
    out_ref[0, 0, 0] = m + jnp.log(acc)


def _tc_part(s_tc, loct_p, w3):
    grid = (B, N_TC // TN)
    out = pl.pallas_call(
        _tc_body,
        grid=grid,
        in_specs=[
            pl.BlockSpec((1, TN, DP), lambda b, t: (b, t, 0)),
            pl.BlockSpec((1, DP, K), lambda b, t: (b, 0, 0)),
            pl.BlockSpec((1, 1, K), lambda b, t: (b, 0, 0)),
        ],
        out_specs=pl.BlockSpec((1, 1, 1, TN), lambda b, t: (b, t, 0, 0)),
        out_shape=jax.ShapeDtypeStruct((B, N_TC // TN, 1, TN), jnp.float32),
    )(s_tc, loct_p, w3)
    return out.reshape(B, N_TC)


@jax.jit
def kernel(sample, loc, weight):
    sample_sc = sample[:, :N_SC]
    sample_tc = sample[:, N_SC:]
    st_sc = (
        sample_sc.reshape(B, CPB, NPW, D)
        .transpose(0, 1, 3, 2)
        .reshape(NW, D * NPW)
    )
    loct = loc.transpose(0, 2, 1)
    loct_flat = loct.reshape(B, D * K)
    loct_p = jnp.pad(loct, [(0, 0), (0, DP - D), (0, 0)])
    s_tc = jnp.pad(sample_tc, [(0, 0), (0, 0), (0, DP - D)])
    w3 = weight.reshape(B, 1, K)

    mesh = plsc.VectorSubcoreMesh(core_axis_name="c", subcore_axis_name="s")
    out_sc = pl.kernel(
        _sc_body,
        out_type=jax.ShapeDtypeStruct((NW, NPW), jnp.float32),
        mesh=mesh,
        scratch_types=[
            pltpu.VMEM((D * K,), jnp.float32),
            pltpu.VMEM((K,), jnp.float32),
            pltpu.VMEM((K,), jnp.float32),
            pltpu.VMEM((D * NPW,), jnp.float32),
            pltpu.VMEM((NPW,), jnp.float32),
        ],
    )(st_sc, loct_flat, weight)

    out_tc = _tc_part(s_tc, loct_p, w3)
    return jnp.concatenate([out_sc.reshape(B, N_SC), out_tc], axis=1)


# speedup vs baseline: 1.0000x; 1.0000x over previous
import functools
import math

import jax
import jax.numpy as jnp
from jax import lax
from jax.experimental import pallas as pl
from jax.experimental.pallas import tpu as pltpu
from jax.experimental.pallas import tpu_sc as plsc

B, N, K, D = 4, 2048, 2048, 4
BW = 0.1
HALF_INV_BW2 = 0.5 / (BW * BW)
INV_BW2 = 1.0 / (BW * BW)
LOGCONST = D * (0.5 * math.log(2.0 * math.pi) + math.log(BW))

NC, NS, L = 2, 16, 16
NW = NC * NS
CPB = NW // B

N_SC = 512
N_TC = N - N_SC
TN = 512

NPW = N_SC // CPB
NV = NPW // L
BLK = min(4, NV)

_SQRT2 = math.sqrt(2.0)
_LN2 = math.log(2.0)


def _vlog(a):
    bits = lax.bitcast_convert_type(a, jnp.int32)
    e = lax.shift_right_logical(bits, 23) - 127
    mant = lax.bitcast_convert_type(
        lax.bitwise_or(lax.bitwise_and(bits, 0x007FFFFF), 0x3F800000),
        jnp.float32,
    )
    big = mant > _SQRT2
    mant = jnp.where(big, mant * 0.5, mant)
    e = jnp.where(big, e + 1, e)
    r = (mant - 1.0) / (mant + 1.0)
    r2 = r * r
    poly = 2.0 * r * (1.0 + r2 * (1.0 / 3.0 + r2 * (1.0 / 5.0 + r2 * (1.0 / 7.0 + r2 * (1.0 / 9.0)))))
    return e.astype(jnp.float32) * _LN2 + poly


def _sc_body(st_ref, loct_ref, w_ref, out_ref, locv, wv, qv, sv, outv):
    wid = lax.axis_index("s") * NC + lax.axis_index("c")
    b = wid // CPB

    pltpu.sync_copy(loct_ref.at[b], locv)
    pltpu.sync_copy(w_ref.at[b], wv)
    pltpu.sync_copy(st_ref.at[wid], sv)

    def qbody(j, _):
        sl = pl.ds(j * L, L)
        tds = [locv[pl.ds(d * K + j * L, L)] for d in range(D)]
        acc = tds[0] * tds[0]
        for d in range(1, D):
            acc = acc + tds[d] * tds[d]
        qv[sl] = wv[sl] - HALF_INV_BW2 * acc - LOGCONST
        for d in range(D):
            locv[pl.ds(d * K + j * L, L)] = INV_BW2 * tds[d]
        return 0

    lax.fori_loop(0, K // L, qbody, 0, unroll=2)

    for blk in range(NV // BLK):
        s_regs = []
        p_regs = []
        for v in range(BLK):
            off = (blk * BLK + v) * L
            sd = [sv[pl.ds(d * NPW + off, L)] for d in range(D)]
            s_regs.append(sd)
            nrm = sd[0] * sd[0]
            for d in range(1, D):
                nrm = nrm + sd[d] * sd[d]
            p_regs.append(-HALF_INV_BW2 * nrm)

        neg = jnp.full((L,), -1e30, dtype=jnp.float32)
        zero = jnp.zeros((L,), dtype=jnp.float32)
        init = tuple([neg] * BLK + [zero] * BLK)

        GRP = 4

        def kbody(j, carry):
            ms = list(carry[:BLK])
            accs = list(carry[BLK:])
            lvecs = [locv[pl.ds(d * K + j * L, L)] for d in range(D)]
            qvec = qv[pl.ds(j * L, L)]
            for g in range(L // GRP):
                xs = [[None] * GRP for _ in range(BLK)]
                for ii in range(GRP):
                    i = g * GRP + ii
                    l = [lvecs[d][i] for d in range(D)]
                    qk = qvec[i]
                    for v in range(BLK):
                        sd = s_regs[v]
                        dot = sd[0] * l[0]
                        for d in range(1, D):
                            dot = dot + sd[d] * l[d]
                        xs[v][ii] = (p_regs[v] + qk) + dot
                for v in range(BLK):
                    mn = ms[v]
                    for ii in range(GRP):
                        mn = jnp.maximum(mn, xs[v][ii])
                    e01 = jnp.exp(xs[v][0] - mn) + jnp.exp(xs[v][1] - mn)
                    e23 = jnp.exp(xs[v][2] - mn) + jnp.exp(xs[v][3] - mn)
                    accs[v] = accs[v] * jnp.exp(ms[v] - mn) + (e01 + e23)
                    ms[v] = mn
            return tuple(ms + accs)

        res = lax.fori_loop(0, K // L, kbody, init, unroll=1)
        for v in range(BLK):
            off = (blk * BLK + v) * L
            outv[pl.ds(off, L)] = res[v] + _vlog(res[BLK + v])

    pltpu.sync_copy(outv, out_ref.at[wid])


DP = 8


def _tc_body(s_ref, loct_ref, w_ref, out_ref):
    s = s_ref[0]
    lt = loct_ref[0]
    q = w_ref[0, 0] - HALF_INV_BW2 * jnp.sum(lt * lt, axis=0) - LOGCONST
    p = -HALF_INV_BW2 * jnp.sum(s * s, axis=1)
    dot = lax.dot_general(
        s,
        lt,
        (((1,), (0,)), ((), ())),
        preferred_element_type=jnp.float32,
        precision=lax.Precision.HIGHEST,
    )
    x = q[None, :] + p[:, None] + INV_BW2 * dot
    m = jnp.max(x, axis=1)
    acc = jnp.sum(jnp.exp(x - m[:, None]), axis=1)
    out_ref[0, 0, 0] = m + jnp.log(acc)


def _tc_part(s_tc, loct_p, w3):
    grid = (B, N_TC // TN)
    out = pl.pallas_call(
        _tc_body,
        grid=grid,
        in_specs=[
            pl.BlockSpec((1, TN, DP), lambda b, t: (b, t, 0)),
            pl.BlockSpec((1, DP, K), lambda b, t: (b, 0, 0)),
            pl.BlockSpec((1, 1, K), lambda b, t: (b, 0, 0)),
        ],
        out_specs=pl.BlockSpec((1, 1, 1, TN), lambda b, t: (b, t, 0, 0)),
        out_shape=jax.ShapeDtypeStruct((B, N_TC // TN, 1, TN), jnp.float32),
    )(s_tc, loct_p, w3)
    return out.reshape(B, N_TC)


@jax.jit
def kernel(sample, loc, weight):
    sample_sc = sample[:, :N_SC]
    sample_tc = sample[:, N_SC:]
    st_sc = (
        sample_sc.reshape(B, CPB, NPW, D)
        .transpose(0, 1, 3, 2)
        .reshape(NW, D * NPW)
    )
    loct = loc.transpose(0, 2, 1)
    loct_flat = loct.reshape(B, D * K)
    loct_p = jnp.pad(loct, [(0, 0), (0, DP - D), (0, 0)])
    s_tc = jnp.pad(sample_tc, [(0, 0), (0, 0), (0, DP - D)])
    w3 = weight.reshape(B, 1, K)

    mesh = plsc.VectorSubcoreMesh(core_axis_name="c", subcore_axis_name="s")
    out_sc = pl.kernel(
        _sc_body,
        out_type=jax.ShapeDtypeStruct((NW, NPW), jnp.float32),
        mesh=mesh,
        scratch_types=[
            pltpu.VMEM((D * K,), jnp.float32),
            pltpu.VMEM((K,), jnp.float32),
            pltpu.VMEM((K,), jnp.float32),
            pltpu.VMEM((D * NPW,), jnp.float32),
            pltpu.VMEM((NPW,), jnp.float32),
        ],
    )(st_sc, loct_flat, weight)

    out_tc = _tc_part(s_tc, loct_p, w3)
    return jnp.concatenate([out_sc.reshape(B, N_SC), out_tc], axis=1)


# speedup vs baseline: 1.3132x; 1.3131x over previous
import functools
import math

import jax
import jax.numpy as jnp
from jax import lax
from jax.experimental import pallas as pl
from jax.experimental.pallas import tpu as pltpu
from jax.experimental.pallas import tpu_sc as plsc

B, N, K, D = 4, 2048, 2048, 4
BW = 0.1
HALF_INV_BW2 = 0.5 / (BW * BW)
INV_BW2 = 1.0 / (BW * BW)
LOGCONST = D * (0.5 * math.log(2.0 * math.pi) + math.log(BW))

NC, NS, L = 2, 16, 16
NW = NC * NS
CPB = NW // B

N_SC = 512
N_TC = N - N_SC
TN = 512

NPW = N_SC // CPB
NV = NPW // L
BLK = min(4, NV)

_SQRT2 = math.sqrt(2.0)
_LN2 = math.log(2.0)


def _vlog(a):
    bits = lax.bitcast_convert_type(a, jnp.int32)
    e = lax.shift_right_logical(bits, 23) - 127
    mant = lax.bitcast_convert_type(
        lax.bitwise_or(lax.bitwise_and(bits, 0x007FFFFF), 0x3F800000),
        jnp.float32,
    )
    big = mant > _SQRT2
    mant = jnp.where(big, mant * 0.5, mant)
    e = jnp.where(big, e + 1, e)
    r = (mant - 1.0) / (mant + 1.0)
    r2 = r * r
    poly = 2.0 * r * (1.0 + r2 * (1.0 / 3.0 + r2 * (1.0 / 5.0 + r2 * (1.0 / 7.0 + r2 * (1.0 / 9.0)))))
    return e.astype(jnp.float32) * _LN2 + poly


def _sc_body(st_ref, loct_ref, w_ref, out_ref, locv, wv, qv, sv, outv):
    wid = lax.axis_index("s") * NC + lax.axis_index("c")
    b = wid // CPB

    pltpu.sync_copy(loct_ref.at[b], locv)
    pltpu.sync_copy(w_ref.at[b], wv)
    pltpu.sync_copy(st_ref.at[wid], sv)

    def qbody(j, _):
        sl = pl.ds(j * L, L)
        tds = [locv[pl.ds(d * K + j * L, L)] for d in range(D)]
        acc = tds[0] * tds[0]
        for d in range(1, D):
            acc = acc + tds[d] * tds[d]
        qv[sl] = wv[sl] - HALF_INV_BW2 * acc - LOGCONST
        for d in range(D):
            locv[pl.ds(d * K + j * L, L)] = INV_BW2 * tds[d]
        return 0

    lax.fori_loop(0, K // L, qbody, 0, unroll=2)

    for blk in range(NV // BLK):
        s_regs = []
        p_regs = []
        for v in range(BLK):
            off = (blk * BLK + v) * L
            sd = [sv[pl.ds(d * NPW + off, L)] for d in range(D)]
            s_regs.append(sd)
            nrm = sd[0] * sd[0]
            for d in range(1, D):
                nrm = nrm + sd[d] * sd[d]
            p_regs.append(-HALF_INV_BW2 * nrm)

        neg = jnp.full((L,), -1e30, dtype=jnp.float32)
        zero = jnp.zeros((L,), dtype=jnp.float32)
        init = tuple([neg] * BLK + [zero] * BLK)

        GRP = 4

        def kbody(j, carry):
            ms = list(carry[:BLK])
            accs = list(carry[BLK:])
            lvecs = [locv[pl.ds(d * K + j * L, L)] for d in range(D)]
            qvec = qv[pl.ds(j * L, L)]
            for g in range(L // GRP):
                xs = [[None] * GRP for _ in range(BLK)]
                for ii in range(GRP):
                    i = g * GRP + ii
                    l = [lvecs[d][i] for d in range(D)]
                    qk = qvec[i]
                    for v in range(BLK):
                        sd = s_regs[v]
                        dot = sd[0] * l[0]
                        for d in range(1, D):
                            dot = dot + sd[d] * l[d]
                        xs[v][ii] = (p_regs[v] + qk) + dot
                for v in range(BLK):
                    mn = ms[v]
                    for ii in range(GRP):
                        mn = jnp.maximum(mn, xs[v][ii])
                    e01 = jnp.exp(xs[v][0] - mn) + jnp.exp(xs[v][1] - mn)
                    e23 = jnp.exp(xs[v][2] - mn) + jnp.exp(xs[v][3] - mn)
                    accs[v] = accs[v] * jnp.exp(ms[v] - mn) + (e01 + e23)
                    ms[v] = mn
            return tuple(ms + accs)

        res = lax.fori_loop(0, K // L, kbody, init, unroll=1)
        for v in range(BLK):
            off = (blk * BLK + v) * L
            outv[pl.ds(off, L)] = res[v] + _vlog(res[BLK + v])

    pltpu.sync_copy(outv, out_ref.at[wid])


DP = 8


def _tc_body(s_ref, loct_ref, w_ref, out_ref):
    s = s_ref[0]
    lt = loct_ref[0]
    q = w_ref[0, 0] - HALF_INV_BW2 * jnp.sum(lt * lt, axis=0) - LOGCONST
    p = -HALF_INV_BW2 * jnp.sum(s * s, axis=1)
    scol = lax.broadcasted_iota(jnp.int32, (TN, DP), 1)
    s_aug = s + jnp.where(scol == D, p[:, None], 0.0) + jnp.where(scol == D + 1, 1.0, 0.0)
    lrow = lax.broadcasted_iota(jnp.int32, (DP, K), 0)
    l_aug = jnp.where(
        lrow == D, 1.0, jnp.where(lrow == D + 1, q[None, :], INV_BW2 * lt)
    )
    dims = (((1,), (0,)), ((), ()))
    sh = s_aug.astype(jnp.bfloat16)
    sl = (s_aug - sh.astype(jnp.float32)).astype(jnp.bfloat16)
    lh = l_aug.astype(jnp.bfloat16)
    ll = (l_aug - lh.astype(jnp.float32)).astype(jnp.bfloat16)
    x = (
        lax.dot_general(sh, lh, dims, preferred_element_type=jnp.float32)
        + lax.dot_general(sh, ll, dims, preferred_element_type=jnp.float32)
        + lax.dot_general(sl, lh, dims, preferred_element_type=jnp.float32)
    )
    m = jnp.max(x, axis=1)
    acc = jnp.sum(jnp.exp(x - m[:, None]), axis=1)
    out_ref[0, 0, 0] = m + jnp.log(acc)


def _tc_part(s_tc, loct_p, w3):
    grid = (B, N_TC // TN)
    out = pl.pallas_call(
        _tc_body,
        grid=grid,
        in_specs=[
            pl.BlockSpec((1, TN, DP), lambda b, t: (b, t, 0)),
            pl.BlockSpec((1, DP, K), lambda b, t: (b, 0, 0)),
            pl.BlockSpec((1, 1, K), lambda b, t: (b, 0, 0)),
        ],
        out_specs=pl.BlockSpec((1, 1, 1, TN), lambda b, t: (b, t, 0, 0)),
        out_shape=jax.ShapeDtypeStruct((B, N_TC // TN, 1, TN), jnp.float32),
    )(s_tc, loct_p, w3)
    return out.reshape(B, N_TC)


@jax.jit
def kernel(sample, loc, weight):
    sample_sc = sample[:, :N_SC]
    sample_tc = sample[:, N_SC:]
    st_sc = (
        sample_sc.reshape(B, CPB, NPW, D)
        .transpose(0, 1, 3, 2)
        .reshape(NW, D * NPW)
    )
    loct = loc.transpose(0, 2, 1)
    loct_flat = loct.reshape(B, D * K)
    loct_p = jnp.pad(loct, [(0, 0), (0, DP - D), (0, 0)])
    s_tc = jnp.pad(sample_tc, [(0, 0), (0, 0), (0, DP - D)])
    w3 = weight.reshape(B, 1, K)

    mesh = plsc.VectorSubcoreMesh(core_axis_name="c", subcore_axis_name="s")
    out_sc = pl.kernel(
        _sc_body,
        out_type=jax.ShapeDtypeStruct((NW, NPW), jnp.float32),
        mesh=mesh,
        scratch_types=[
            pltpu.VMEM((D * K,), jnp.float32),
            pltpu.VMEM((K,), jnp.float32),
            pltpu.VMEM((K,), jnp.float32),
            pltpu.VMEM((D * NPW,), jnp.float32),
            pltpu.VMEM((NPW,), jnp.float32),
        ],
    )(st_sc, loct_flat, weight)

    out_tc = _tc_part(s_tc, loct_p, w3)
    return jnp.concatenate([out_sc.reshape(B, N_SC), out_tc], axis=1)


# speedup vs baseline: 1.5125x; 1.1518x over previous
import functools
import math

import jax
import jax.numpy as jnp
from jax import lax
from jax.experimental import pallas as pl
from jax.experimental.pallas import tpu as pltpu
from jax.experimental.pallas import tpu_sc as plsc

B, N, K, D = 4, 2048, 2048, 4
BW = 0.1
HALF_INV_BW2 = 0.5 / (BW * BW)
INV_BW2 = 1.0 / (BW * BW)
LOGCONST = D * (0.5 * math.log(2.0 * math.pi) + math.log(BW))

NC, NS, L = 2, 16, 16
NW = NC * NS
CPB = NW // B

N_SC = 512
N_TC = N - N_SC
TN = 768

NPW = N_SC // CPB
NV = NPW // L
BLK = min(4, NV)

_SQRT2 = math.sqrt(2.0)
_LN2 = math.log(2.0)


def _vlog(a):
    bits = lax.bitcast_convert_type(a, jnp.int32)
    e = lax.shift_right_logical(bits, 23) - 127
    mant = lax.bitcast_convert_type(
        lax.bitwise_or(lax.bitwise_and(bits, 0x007FFFFF), 0x3F800000),
        jnp.float32,
    )
    big = mant > _SQRT2
    mant = jnp.where(big, mant * 0.5, mant)
    e = jnp.where(big, e + 1, e)
    r = (mant - 1.0) / (mant + 1.0)
    r2 = r * r
    poly = 2.0 * r * (1.0 + r2 * (1.0 / 3.0 + r2 * (1.0 / 5.0 + r2 * (1.0 / 7.0 + r2 * (1.0 / 9.0)))))
    return e.astype(jnp.float32) * _LN2 + poly


def _sc_body(st_ref, loct_ref, w_ref, out_ref, locv, wv, qv, sv, outv):
    wid = lax.axis_index("s") * NC + lax.axis_index("c")
    b = wid // CPB

    pltpu.sync_copy(loct_ref.at[b], locv)
    pltpu.sync_copy(w_ref.at[b], wv)
    pltpu.sync_copy(st_ref.at[wid], sv)

    def qbody(j, _):
        sl = pl.ds(j * L, L)
        tds = [locv[pl.ds(d * K + j * L, L)] for d in range(D)]
        acc = tds[0] * tds[0]
        for d in range(1, D):
            acc = acc + tds[d] * tds[d]
        qv[sl] = wv[sl] - HALF_INV_BW2 * acc - LOGCONST
        for d in range(D):
            locv[pl.ds(d * K + j * L, L)] = INV_BW2 * tds[d]
        return 0

    lax.fori_loop(0, K // L, qbody, 0, unroll=2)

    for blk in range(NV // BLK):
        s_regs = []
        p_regs = []
        for v in range(BLK):
            off = (blk * BLK + v) * L
            sd = [sv[pl.ds(d * NPW + off, L)] for d in range(D)]
            s_regs.append(sd)
            nrm = sd[0] * sd[0]
            for d in range(1, D):
                nrm = nrm + sd[d] * sd[d]
            p_regs.append(-HALF_INV_BW2 * nrm)

        neg = jnp.full((L,), -1e30, dtype=jnp.float32)
        zero = jnp.zeros((L,), dtype=jnp.float32)
        init = tuple([neg] * BLK + [zero] * BLK)

        GRP = 4

        def kbody(j, carry):
            ms = list(carry[:BLK])
            accs = list(carry[BLK:])
            lvecs = [locv[pl.ds(d * K + j * L, L)] for d in range(D)]
            qvec = qv[pl.ds(j * L, L)]
            for g in range(L // GRP):
                xs = [[None] * GRP for _ in range(BLK)]
                for ii in range(GRP):
                    i = g * GRP + ii
                    l = [lvecs[d][i] for d in range(D)]
                    qk = qvec[i]
                    for v in range(BLK):
                        sd = s_regs[v]
                        dot = sd[0] * l[0]
                        for d in range(1, D):
                            dot = dot + sd[d] * l[d]
                        xs[v][ii] = (p_regs[v] + qk) + dot
                for v in range(BLK):
                    mn = ms[v]
                    for ii in range(GRP):
                        mn = jnp.maximum(mn, xs[v][ii])
                    e01 = jnp.exp(xs[v][0] - mn) + jnp.exp(xs[v][1] - mn)
                    e23 = jnp.exp(xs[v][2] - mn) + jnp.exp(xs[v][3] - mn)
                    accs[v] = accs[v] * jnp.exp(ms[v] - mn) + (e01 + e23)
                    ms[v] = mn
            return tuple(ms + accs)

        res = lax.fori_loop(0, K // L, kbody, init, unroll=1)
        for v in range(BLK):
            off = (blk * BLK + v) * L
            outv[pl.ds(off, L)] = res[v] + _vlog(res[BLK + v])

    pltpu.sync_copy(outv, out_ref.at[wid])


DP = 8


def _tc_body(s_ref, loct_ref, w_ref, out_ref):
    s = s_ref[0]
    lt = loct_ref[0]
    q = w_ref[0, 0] - HALF_INV_BW2 * jnp.sum(lt * lt, axis=0) - LOGCONST
    p = -HALF_INV_BW2 * jnp.sum(s * s, axis=1)
    scol = lax.broadcasted_iota(jnp.int32, (TN, DP), 1)
    s_aug = s + jnp.where(scol == D, p[:, None], 0.0) + jnp.where(scol == D + 1, 1.0, 0.0)
    lrow = lax.broadcasted_iota(jnp.int32, (DP, K), 0)
    l_aug = jnp.where(
        lrow == D, 1.0, jnp.where(lrow == D + 1, q[None, :], INV_BW2 * lt)
    )
    dims = (((1,), (0,)), ((), ()))
    sh = s_aug.astype(jnp.bfloat16)
    sl = (s_aug - sh.astype(jnp.float32)).astype(jnp.bfloat16)
    lh = l_aug.astype(jnp.bfloat16)
    ll = (l_aug - lh.astype(jnp.float32)).astype(jnp.bfloat16)
    s_cat = jnp.concatenate([sh, sh, sl], axis=1)
    l_cat = jnp.concatenate([lh, ll, lh], axis=0)
    x = lax.dot_general(
        s_cat, l_cat, dims, preferred_element_type=jnp.float32
    )
    m = jnp.max(x, axis=1)
    acc = jnp.sum(jnp.exp(x - m[:, None]), axis=1)
    out_ref[0, 0, 0] = m + jnp.log(acc)


def _tc_part(s_tc, loct_p, w3):
    grid = (B, N_TC // TN)
    out = pl.pallas_call(
        _tc_body,
        grid=grid,
        in_specs=[
            pl.BlockSpec((1, TN, DP), lambda b, t: (b, t, 0)),
            pl.BlockSpec((1, DP, K), lambda b, t: (b, 0, 0)),
            pl.BlockSpec((1, 1, K), lambda b, t: (b, 0, 0)),
        ],
        out_specs=pl.BlockSpec((1, 1, 1, TN), lambda b, t: (b, t, 0, 0)),
        out_shape=jax.ShapeDtypeStruct((B, N_TC // TN, 1, TN), jnp.float32),
    )(s_tc, loct_p, w3)
    return out.reshape(B, N_TC)


@jax.jit
def kernel(sample, loc, weight):
    sample_sc = sample[:, :N_SC]
    sample_tc = sample[:, N_SC:]
    st_sc = (
        sample_sc.reshape(B, CPB, NPW, D)
        .transpose(0, 1, 3, 2)
        .reshape(NW, D * NPW)
    )
    loct = loc.transpose(0, 2, 1)
    loct_flat = loct.reshape(B, D * K)
    loct_p = jnp.pad(loct, [(0, 0), (0, DP - D), (0, 0)])
    s_tc = jnp.pad(sample_tc, [(0, 0), (0, 0), (0, DP - D)])
    w3 = weight.reshape(B, 1, K)

    mesh = plsc.VectorSubcoreMesh(core_axis_name="c", subcore_axis_name="s")
    out_sc = pl.kernel(
        _sc_body,
        out_type=jax.ShapeDtypeStruct((NW, NPW), jnp.float32),
        mesh=mesh,
        scratch_types=[
            pltpu.VMEM((D * K,), jnp.float32),
            pltpu.VMEM((K,), jnp.float32),
            pltpu.VMEM((K,), jnp.float32),
            pltpu.VMEM((D * NPW,), jnp.float32),
            pltpu.VMEM((NPW,), jnp.float32),
        ],
    )(st_sc, loct_flat, weight)

    out_tc = _tc_part(s_tc, loct_p, w3)
    return jnp.concatenate([out_sc.reshape(B, N_SC), out_tc], axis=1)
